# baseline (speedup 1.0000x reference)
import functools

import jax
import jax.numpy as jnp
from jax import lax
from jax.experimental import pallas as pl
from jax.experimental.pallas import tpu as pltpu
from jax.experimental.pallas import tpu_sc as plsc

_L = 14
_P = _L * _L
_R = 5
_THRESHOLD = 0.6


def _combine(a, b):
    av, ai = a
    bv, bi = b
    repl = (bv > av) | ((bv == av) & (bi < ai))
    return jnp.where(repl, bv, av), jnp.where(repl, bi, ai)


def _nms_body(bpw, h_hbm, out1_hbm, out2_hbm, heat_v, out1_v, out2_v):
    info = plsc.get_sparse_core_info()
    nc, lanes_n = info.num_cores, info.num_lanes
    chunk = bpw * _P
    ngroups = bpw // lanes_n

    wid = lax.axis_index("s") * nc + lax.axis_index("c")
    base = wid * chunk
    pltpu.sync_copy(h_hbm.at[pl.ds(base, chunk)], heat_v)

    lanes = lax.iota(jnp.int32, lanes_n)
    zeros_f = jnp.zeros((lanes_n,), jnp.float32)
    ones_f = jnp.full((lanes_n,), 1.0, jnp.float32)

    def group_body(g, carry):
        bvec = (g * lanes_n + lanes) * _P

        ims = []
        ci_glob = bvec
        for r in range(4):

            def scan_rows(i, accs):
                row = bvec + i * _L
                out = []
                for j in range(_L):
                    cm, ci = accs[j]
                    idx = row + j
                    v = plsc.load_gather(heat_v, [idx])
                    cond = v > cm
                    out.append((
                        jnp.where(cond, v, cm),
                        jnp.where(cond, idx, ci),
                    ))
                return tuple(out)

            init = tuple(
                (jnp.full((lanes_n,), _THRESHOLD, jnp.float32), bvec)
                for _ in range(_L)
            )
            accs = lax.fori_loop(0, _L, scan_rows, init)
            accs = list(accs)
            while len(accs) > 1:
                nxt = [
                    _combine(accs[2 * t], accs[2 * t + 1])
                    for t in range(len(accs) // 2)
                ]
                if len(accs) % 2:
                    nxt.append(accs[-1])
                accs = nxt
            _, ci_glob = accs[0]
            im = ci_glob - bvec
            ims.append(im)

            if r < 3:
                x = im // _L
                y = im - x * _L
                okys = []
                for dyj in range(2 * _R):
                    yn = y + (dyj - _R)
                    okys.append((yn >= 0) & (yn < _L))

                def sup_body(t, ci):
                    dx = t - _R
                    xn = x + dx
                    okx = (xn >= 0) & (xn < _L)
                    row_t = ci + dx * _L
                    for dyj in range(2 * _R):
                        ok = okx & okys[dyj]
                        tgt = row_t + (dyj - _R)
                        plsc.store_scatter(heat_v, [tgt], zeros_f, mask=ok)
                    return ci

                lax.fori_loop(0, 2 * _R, sup_body, ci_glob)

        xs = [im // _L for im in ims]
        ys = [im - (im // _L) * _L for im in ims]
        pairs = [(0, 1), (0, 2), (0, 3), (1, 2), (1, 3), (2, 3)]
        best = jnp.full((lanes_n,), -1, jnp.int32)
        c1x, c1y, c2x, c2y = xs[0], ys[0], xs[1], ys[1]
        for a, b in pairs:
            dxx = xs[b] - xs[a]
            dyy = ys[b] - ys[a]
            d = dxx * dxx + dyy * dyy
            cond = d > best
            best = jnp.where(cond, d, best)
            c1x = jnp.where(cond, xs[a], c1x)
            c1y = jnp.where(cond, ys[a], c1y)
            c2x = jnp.where(cond, xs[b], c2x)
            c2y = jnp.where(cond, ys[b], c2y)

        ax = 2 * (c2x - c1x)
        ay = 2 * (c2y - c1y)
        kk = c2x * c2x + c2y * c2y - c1x * c1x - c1y * c1y
        jays = [ay * j for j in range(_L)]

        def vor_rows(i, c):
            rbase = i * ax - kk
            row = bvec + i * _L
            for j in range(_L):
                lhs = rbase + jays[j] if j else rbase
                m = lhs < 0
                m1 = jnp.where(m, ones_f, zeros_f)
                m2 = ones_f - m1
                tgt = row + j
                plsc.store_scatter(out1_v, [tgt], m1)
                plsc.store_scatter(out2_v, [tgt], m2)
            return c

        lax.fori_loop(0, _L, vor_rows, 0)
        return carry

    lax.fori_loop(0, ngroups, group_body, 0)

    pltpu.sync_copy(out1_v, out1_hbm.at[pl.ds(base, chunk)])
    pltpu.sync_copy(out2_v, out2_hbm.at[pl.ds(base, chunk)])


@functools.partial(jax.jit, static_argnums=(1,))
def _nms_run(hflat, bpw):
    chunk = bpw * _P
    n = hflat.shape[0]
    mesh = plsc.VectorSubcoreMesh(core_axis_name="c", subcore_axis_name="s")
    out = pl.kernel(
        functools.partial(_nms_body, bpw),
        out_type=(
            jax.ShapeDtypeStruct((n,), jnp.float32),
            jax.ShapeDtypeStruct((n,), jnp.float32),
        ),
        mesh=mesh,
        compiler_params=pltpu.CompilerParams(needs_layout_passes=False),
        scratch_types=[
            pltpu.VMEM((chunk,), jnp.float32),
            pltpu.VMEM((chunk,), jnp.float32),
            pltpu.VMEM((chunk,), jnp.float32),
        ],
    )(hflat)
    return out


def kernel(heatmap):
    b = heatmap.shape[0]
    info = plsc.get_sparse_core_info()
    nw = info.num_cores * info.num_subcores
    bpw = b // nw
    hflat = heatmap.reshape(b * _P)
    o1, o2 = _nms_run(hflat, bpw)
    return (o1.reshape(b, 1, _L, _L), o2.reshape(b, 1, _L, _L))


# speedup vs baseline: 1.0014x; 1.0014x over previous
import functools

import jax
import jax.numpy as jnp
from jax import lax
from jax.experimental import pallas as pl
from jax.experimental.pallas import tpu as pltpu
from jax.experimental.pallas import tpu_sc as plsc

_L = 14
_P = _L * _L
_R = 5
_THRESHOLD = 0.6
_C = 128


def _combine(a, b):
    av, ai = a
    bv, bi = b
    repl = (bv > av) | ((bv == av) & (bi < ai))
    return jnp.where(repl, bv, av), jnp.where(repl, bi, ai)


def _nms_body(bpw, h_hbm, out1_hbm, out2_hbm, heat_v, out1_v, out2_v):
    info = plsc.get_sparse_core_info()
    nc, lanes_n = info.num_cores, info.num_lanes
    rows = bpw * _P // _C
    ngroups = bpw // lanes_n

    wid = lax.axis_index("s") * nc + lax.axis_index("c")
    pltpu.sync_copy(h_hbm.at[wid], heat_v)

    lanes = lax.iota(jnp.int32, lanes_n)
    zeros_f = jnp.zeros((lanes_n,), jnp.float32)
    ones_f = jnp.full((lanes_n,), 1.0, jnp.float32)

    def gat(ref, flat):
        return plsc.load_gather(ref, [flat >> 7, flat & (_C - 1)])

    def scat(ref, flat, val, mask=None):
        plsc.store_scatter(ref, [flat >> 7, flat & (_C - 1)], val, mask=mask)

    def group_body(g, carry):
        bvec = (g * lanes_n + lanes) * _P

        ims = []
        ci_glob = bvec
        for r in range(4):

            def scan_rows(i, accs):
                row = bvec + i * _L
                out = []
                for j in range(_L):
                    cm, ci = accs[j]
                    idx = row + j
                    v = gat(heat_v, idx)
                    cond = v > cm
                    out.append((
                        jnp.where(cond, v, cm),
                        jnp.where(cond, idx, ci),
                    ))
                return tuple(out)

            init = tuple(
                (jnp.full((lanes_n,), _THRESHOLD, jnp.float32), bvec)
                for _ in range(_L)
            )
            accs = lax.fori_loop(0, _L, scan_rows, init)
            accs = list(accs)
            while len(accs) > 1:
                nxt = [
                    _combine(accs[2 * t], accs[2 * t + 1])
                    for t in range(len(accs) // 2)
                ]
                if len(accs) % 2:
                    nxt.append(accs[-1])
                accs = nxt
            _, ci_glob = accs[0]
            im = ci_glob - bvec
            ims.append(im)

            if r < 3:
                x = im // _L
                y = im - x * _L
                okys = []
                for dyj in range(2 * _R):
                    yn = y + (dyj - _R)
                    okys.append((yn >= 0) & (yn < _L))

                def sup_body(t, ci):
                    dx = t - _R
                    xn = x + dx
                    okx = (xn >= 0) & (xn < _L)
                    row_t = ci + dx * _L
                    for dyj in range(2 * _R):
                        ok = okx & okys[dyj]
                        tgt = row_t + (dyj - _R)
                        scat(heat_v, tgt, zeros_f, mask=ok)
                    return ci

                lax.fori_loop(0, 2 * _R, sup_body, ci_glob)

        xs = [im // _L for im in ims]
        ys = [im - (im // _L) * _L for im in ims]
        pairs = [(0, 1), (0, 2), (0, 3), (1, 2), (1, 3), (2, 3)]
        best = jnp.full((lanes_n,), -1, jnp.int32)
        c1x, c1y, c2x, c2y = xs[0], ys[0], xs[1], ys[1]
        for a, b in pairs:
            dxx = xs[b] - xs[a]
            dyy = ys[b] - ys[a]
            d = dxx * dxx + dyy * dyy
            cond = d > best
            best = jnp.where(cond, d, best)
            c1x = jnp.where(cond, xs[a], c1x)
            c1y = jnp.where(cond, ys[a], c1y)
            c2x = jnp.where(cond, xs[b], c2x)
            c2y = jnp.where(cond, ys[b], c2y)

        ax = 2 * (c2x - c1x)
        ay = 2 * (c2y - c1y)
        kk = c2x * c2x + c2y * c2y - c1x * c1x - c1y * c1y
        jays = [ay * j for j in range(_L)]

        def vor_rows(i, c):
            rbase = i * ax - kk
            row = bvec + i * _L
            for j in range(_L):
                lhs = rbase + jays[j] if j else rbase
                m = lhs < 0
                m1 = jnp.where(m, ones_f, zeros_f)
                m2 = ones_f - m1
                tgt = row + j
                scat(out1_v, tgt, m1)
                scat(out2_v, tgt, m2)
            return c

        lax.fori_loop(0, _L, vor_rows, 0)
        return carry

    lax.fori_loop(0, ngroups, group_body, 0)

    pltpu.sync_copy(out1_v, out1_hbm.at[wid])
    pltpu.sync_copy(out2_v, out2_hbm.at[wid])


@functools.partial(jax.jit, static_argnums=(1,))
def _nms_run(h3, bpw):
    nw = h3.shape[0]
    rows = h3.shape[1]
    mesh = plsc.VectorSubcoreMesh(core_axis_name="c", subcore_axis_name="s")
    out = pl.kernel(
        functools.partial(_nms_body, bpw),
        out_type=(
            jax.ShapeDtypeStruct((nw, rows, _C), jnp.float32),
            jax.ShapeDtypeStruct((nw, rows, _C), jnp.float32),
        ),
        mesh=mesh,
        compiler_params=pltpu.CompilerParams(needs_layout_passes=False),
        scratch_types=[
            pltpu.VMEM((rows, _C), jnp.float32),
            pltpu.VMEM((rows, _C), jnp.float32),
            pltpu.VMEM((rows, _C), jnp.float32),
        ],
    )(h3)
    return out


def kernel(heatmap):
    b = heatmap.shape[0]
    info = plsc.get_sparse_core_info()
    nw = info.num_cores * info.num_subcores
    bpw = b // nw
    rows = bpw * _P // _C
    h3 = heatmap.reshape(nw, rows, _C)
    o1, o2 = _nms_run(h3, bpw)
    return (
        o1.reshape(b, 1, _L, _L),
        o2.reshape(b, 1, _L, _L),
    )


# speedup vs baseline: 2.0987x; 2.0957x over previous
import functools

import jax
import jax.numpy as jnp
from jax import lax
from jax.experimental import pallas as pl
from jax.experimental.pallas import tpu as pltpu
from jax.experimental.pallas import tpu_sc as plsc

_L = 14
_P = _L * _L
_R = 5
_THRESHOLD = 0.6
_CIO = 256
_SH = 197


def _combine(a, b):
    av, ai = a
    bv, bi = b
    repl = (bv > av) | ((bv == av) & (bi < ai))
    return jnp.where(repl, bv, av), jnp.where(repl, bi, ai)


def _nms_body(bpw, h_hbm, out1_hbm, out2_hbm, land1, land2, heat_v):
    info = plsc.get_sparse_core_info()
    nc, lanes_n = info.num_cores, info.num_lanes
    ngroups = bpw // lanes_n

    wid = lax.axis_index("s") * nc + lax.axis_index("c")
    pltpu.sync_copy(h_hbm.at[wid], land1)

    lanes = lax.iota(jnp.int32, lanes_n)

    nchunk = (_P + lanes_n - 1) // lanes_n

    def repack(e, c):
        dst = e * _SH
        for k in range(nchunk):
            v = land1[e, pl.ds(k * lanes_n, lanes_n)]
            plsc.store_scatter(heat_v, [dst + k * lanes_n + lanes], v)
        return c

    lax.fori_loop(0, bpw, repack, 0)

    zeros_f = jnp.zeros((lanes_n,), jnp.float32)
    ones_f = jnp.full((lanes_n,), 1.0, jnp.float32)

    u0 = 13 * lanes
    iv0 = u0 // _L
    jv0 = u0 - iv0 * _L

    def group_body(g, carry):
        exv = g * lanes_n + lanes
        bvec = exv * _SH

        ims = []
        ci_glob = bvec
        for r in range(4):

            def scan_rows(i, accs):
                row = bvec + i * _L
                out = []
                for j in range(_L):
                    cm, ci = accs[j]
                    idx = row + j
                    v = plsc.load_gather(heat_v, [idx])
                    cond = v > cm
                    out.append((
                        jnp.where(cond, v, cm),
                        jnp.where(cond, idx, ci),
                    ))
                return tuple(out)

            init = tuple(
                (jnp.full((lanes_n,), _THRESHOLD, jnp.float32), bvec)
                for _ in range(_L)
            )
            accs = lax.fori_loop(0, _L, scan_rows, init)
            accs = list(accs)
            while len(accs) > 1:
                nxt = [
                    _combine(accs[2 * t], accs[2 * t + 1])
                    for t in range(len(accs) // 2)
                ]
                if len(accs) % 2:
                    nxt.append(accs[-1])
                accs = nxt
            _, ci_glob = accs[0]
            im = ci_glob - bvec
            ims.append(im)

            if r < 3:
                x = im // _L
                y = im - x * _L
                okys = []
                for dyj in range(2 * _R):
                    yn = y + (dyj - _R)
                    okys.append((yn >= 0) & (yn < _L))

                def sup_body(t, ci):
                    dx = t - _R
                    xn = x + dx
                    okx = (xn >= 0) & (xn < _L)
                    row_t = ci + dx * _L
                    for dyj in range(2 * _R):
                        ok = okx & okys[dyj]
                        tgt = row_t + (dyj - _R)
                        plsc.store_scatter(heat_v, [tgt], zeros_f, mask=ok)
                    return ci

                lax.fori_loop(0, 2 * _R, sup_body, ci_glob)

        xs = [im // _L for im in ims]
        ys = [im - (im // _L) * _L for im in ims]
        pairs = [(0, 1), (0, 2), (0, 3), (1, 2), (1, 3), (2, 3)]
        best = jnp.full((lanes_n,), -1, jnp.int32)
        c1x, c1y, c2x, c2y = xs[0], ys[0], xs[1], ys[1]
        for a, b in pairs:
            dxx = xs[b] - xs[a]
            dyy = ys[b] - ys[a]
            d = dxx * dxx + dyy * dyy
            cond = d > best
            best = jnp.where(cond, d, best)
            c1x = jnp.where(cond, xs[a], c1x)
            c1y = jnp.where(cond, ys[a], c1y)
            c2x = jnp.where(cond, xs[b], c2x)
            c2y = jnp.where(cond, ys[b], c2y)

        ax = 2 * (c2x - c1x)
        ay = 2 * (c2y - c1y)
        kk = c2x * c2x + c2y * c2y - c1x * c1x - c1y * c1y

        def vor_block(s, st):
            iv, jv = st
            for _ in range(_L):
                lhs = iv * ax + jv * ay - kk
                m = lhs < 0
                m1 = jnp.where(m, ones_f, zeros_f)
                m2 = ones_f - m1
                pv = iv * _L + jv
                plsc.store_scatter(land1, [exv, pv], m1)
                plsc.store_scatter(land2, [exv, pv], m2)
                jz = jv == 0
                jv = jnp.where(jz, _L - 1, jv - 1)
                iv = jnp.where(jz, iv, iv + 1)
                iv = jnp.where(iv >= _L, iv - _L, iv)
            return iv, jv

        lax.fori_loop(0, _L, vor_block, (iv0, jv0))
        return carry

    lax.fori_loop(0, ngroups, group_body, 0)

    pltpu.sync_copy(land1, out1_hbm.at[wid])
    pltpu.sync_copy(land2, out2_hbm.at[wid])


@functools.partial(jax.jit, static_argnums=(1,))
def _nms_run(h3, bpw):
    nw = h3.shape[0]
    info = plsc.get_sparse_core_info()
    mesh = plsc.VectorSubcoreMesh(core_axis_name="c", subcore_axis_name="s")
    out = pl.kernel(
        functools.partial(_nms_body, bpw),
        out_type=(
            jax.ShapeDtypeStruct((nw, bpw, _CIO), jnp.float32),
            jax.ShapeDtypeStruct((nw, bpw, _CIO), jnp.float32),
        ),
        mesh=mesh,
        compiler_params=pltpu.CompilerParams(needs_layout_passes=False),
        scratch_types=[
            pltpu.VMEM((bpw, _CIO), jnp.float32),
            pltpu.VMEM((bpw, _CIO), jnp.float32),
            pltpu.VMEM((bpw * _SH + info.num_lanes,), jnp.float32),
        ],
    )(h3)
    return out


def kernel(heatmap):
    b = heatmap.shape[0]
    info = plsc.get_sparse_core_info()
    nw = info.num_cores * info.num_subcores
    bpw = b // nw
    h2 = jnp.pad(heatmap.reshape(b, _P), ((0, 0), (0, _CIO - _P)))
    o1, o2 = _nms_run(h2.reshape(nw, bpw, _CIO), bpw)
    o1 = o1.reshape(b, _CIO)[:, :_P].reshape(b, 1, _L, _L)
    o2 = o2.reshape(b, _CIO)[:, :_P].reshape(b, 1, _L, _L)
    return (o1, o2)
